# baseline (speedup 1.0000x reference)
import jax
import jax.numpy as jnp
from jax import lax
from jax.experimental import pallas as pl
from jax.experimental.pallas import tpu as pltpu
from jax.experimental.pallas import tpu_sc as plsc

B = 4096
W = 50
NEG = 5
D = 64
DP = 128
NUM_SEQ = 100000
NUM_ITEM = 1000000
NT = 4
NROW = W + 1 + NEG
NMET = NROW * NT
NLANE = 16
NC, NS = 2, 16
NWORK = NC * NS
NB = B // NWORK
KCH = D // NLANE


def _bc_i(s):
    return jnp.zeros((NLANE,), jnp.int32) + s


def _bc_f(s):
    return jnp.zeros((NLANE,), jnp.float32) + s


def _build_flat_meta_idx(idx_ref, base, fidx_ref, n, stride):
    iota = lax.iota(jnp.int32, NLANE)
    for j in range(n * NT // NLANE):
        f = iota + j * NLANE
        it = plsc.load_gather(idx_ref, [base + lax.shift_right_logical(f, 2)])
        fidx_ref[pl.ds(j * NLANE, NLANE)] = it + lax.bitwise_and(f, 3) * stride


def _chunked_igather(tab_hbm, idx_ref, dst_ref, sem, n, chunk=112):
    descs = []
    for c in range(0, n, chunk):
        m = min(chunk, n - c)
        descs.append(pltpu.async_copy(
            tab_hbm.at[idx_ref.at[pl.ds(c, m)]], dst_ref.at[pl.ds(c, m)], sem))
    return descs


def _combine_rows(rows_ref, mrows_ref, mw_ref, n, unroll):
    @plsc.parallel_loop(0, n, unroll=unroll)
    def _(i):
        wts = [plsc.load_gather(mw_ref, [_bc_i(i * NT + t)]) for t in range(NT)]
        for k in range(KCH):
            sl = pl.ds(k * NLANE, NLANE)
            acc = rows_ref[i, sl]
            for t in range(NT):
                acc = acc + wts[t] * mrows_ref[i * NT + t, sl]
            rows_ref[i, sl] = acc * jnp.float32(1.0 / (NT + 1))


def _sc_meta_body(idx_all, seq_index, seq_elem, seq_meta_emb,
                  item_meta_emb, seq_mi, seq_mw, item_mi, item_mw,
                  u_out, msum_out,
                  sidx_v, sfidx_v, srows_v, smi_v, smw_v, smrows_v,
                  idxa_v,
                  ifidx0, mi0, mw0, mrows0, msv0,
                  ifidx1, mi1, mw1, mrows1, msv1,
                  sem1, sem2, sem3,
                  semI0, semW0, semM0, semB0, semI1, semW1, semM1, semB1):
    wid = lax.axis_index("s") * NC + lax.axis_index("c")
    base = wid * NB

    pltpu.sync_copy(seq_index.at[pl.ds(base, NB)], sidx_v)
    c_e = pltpu.async_copy(seq_elem.at[sidx_v], srows_v, sem1)
    _build_flat_meta_idx(sidx_v, 0, sfidx_v, NB, NUM_SEQ)
    dmi = _chunked_igather(seq_mi, sfidx_v, smi_v, sem2, NB * NT, 128)
    dmw = _chunked_igather(seq_mw, sfidx_v, smw_v, sem2, NB * NT, 128)
    for dcp in dmi:
        dcp.wait()
    dmr = _chunked_igather(seq_meta_emb, smi_v, smrows_v, sem3, NB * NT, 128)
    for dcp in dmw + dmr:
        dcp.wait()
    c_e.wait()
    _combine_rows(srows_v, smrows_v, smw_v, NB, 8)
    pltpu.sync_copy(srows_v, u_out.at[pl.ds(base, NB)])

    pltpu.sync_copy(idx_all.at[pl.ds(base * NROW, NB * NROW)], idxa_v)

    slots = (
        dict(ifidx=ifidx0, mi=mi0, mw=mw0, mrows=mrows0, msv=msv0,
             semI=semI0, semW=semW0, semM=semM0, semB=semB0),
        dict(ifidx=ifidx1, mi=mi1, mw=mw1, mrows=mrows1, msv=msv1,
             semI=semI1, semW=semW1, semM=semM1, semB=semB1),
    )

    def fire_mi(b, sl):
        _build_flat_meta_idx(idxa_v, b * NROW, sl['ifidx'], NROW, NUM_ITEM)
        _chunked_igather(item_mi, sl['ifidx'], sl['mi'], sl['semI'], NMET)

    def wait_mi(sl):
        for c in range(0, NMET, 112):
            pltpu.make_async_copy(
                item_mi.at[sl['ifidx'].at[pl.ds(c, 112)]],
                sl['mi'].at[pl.ds(c, 112)], sl['semI']).wait()

    def fire_mw(sl):
        _chunked_igather(item_mw, sl['ifidx'], sl['mw'], sl['semW'], NMET)

    def wait_mw(sl):
        for c in range(0, NMET, 112):
            pltpu.make_async_copy(
                item_mw.at[sl['ifidx'].at[pl.ds(c, 112)]],
                sl['mw'].at[pl.ds(c, 112)], sl['semW']).wait()

    def fire_mrows(sl):
        _chunked_igather(item_meta_emb, sl['mi'], sl['mrows'], sl['semM'], NMET)

    def wait_mrows(sl):
        for c in range(0, NMET, 112):
            pltpu.make_async_copy(
                item_meta_emb.at[sl['mi'].at[pl.ds(c, 112)]],
                sl['mrows'].at[pl.ds(c, 112)], sl['semM']).wait()

    def wb_desc(b, sl):
        return pltpu.make_async_copy(
            sl['msv'], msum_out.at[pl.ds((base + b) * NROW, NROW)], sl['semB'])

    def compute_msum(sl):
        mrows_ref, mw_ref, msv_ref = sl['mrows'], sl['mw'], sl['msv']

        @plsc.parallel_loop(0, NROW, unroll=8)
        def _(i):
            wts = [plsc.load_gather(mw_ref, [_bc_i(i * NT + t)])
                   for t in range(NT)]
            for k in range(KCH):
                sl2 = pl.ds(k * NLANE, NLANE)
                acc = wts[0] * mrows_ref[i * NT, sl2]
                for t in range(1, NT):
                    acc = acc + wts[t] * mrows_ref[i * NT + t, sl2]
                msv_ref[i, sl2] = acc

    fire_mi(0, slots[0])
    fire_mw(slots[0])
    fire_mi(1, slots[1])
    fire_mw(slots[1])
    wait_mi(slots[0])
    fire_mrows(slots[0])

    def gbody(g, carry):
        for j in range(2):
            b = 2 * g + j
            s, o = slots[j], slots[1 - j]

            @pl.when(b + 1 < NB)
            def _():
                wait_mi(o)
                fire_mrows(o)

            wait_mw(s)

            @pl.when(b + 2 < NB)
            def _():
                fire_mi(b + 2, s)

            wait_mrows(s)

            @pl.when(b >= 2)
            def _():
                wb_desc(b - 2, s).wait()

            compute_msum(s)
            pltpu.async_copy(
                s['msv'], msum_out.at[pl.ds((base + b) * NROW, NROW)],
                s['semB'])

            @pl.when(b + 2 < NB)
            def _():
                fire_mw(s)
        return carry
    lax.fori_loop(0, NB // 2, gbody, 0)
    wb_desc(NB - 2, slots[0]).wait()
    wb_desc(NB - 1, slots[1]).wait()


def _sc_attn_body(idx_all, u_tab, item_elem, msum,
                  out_hbm,
                  idxa_v, u_v, scores_v, out_v,
                  erows0, msb0, erows1, msb1,
                  semE0, semS0, semE1, semS1):
    wid = lax.axis_index("s") * NC + lax.axis_index("c")
    base = wid * NB
    iota = lax.iota(jnp.int32, NLANE)
    lane0 = iota == 0

    pltpu.sync_copy(idx_all.at[pl.ds(base * NROW, NB * NROW)], idxa_v)
    pltpu.sync_copy(u_tab.at[pl.ds(base, NB)], u_v)

    slots = (
        dict(erows=erows0, msb=msb0, semE=semE0, semS=semS0),
        dict(erows=erows1, msb=msb1, semE=semE1, semS=semS1),
    )

    def fire_eb(b, sl):
        pltpu.async_copy(item_elem.at[idxa_v.at[pl.ds(b * NROW, NROW)]],
                         sl['erows'], sl['semE'])
        pltpu.async_copy(msum.at[pl.ds((base + b) * NROW, NROW)],
                         sl['msb'], sl['semS'])

    def wait_eb(b, sl):
        pltpu.make_async_copy(item_elem.at[idxa_v.at[pl.ds(b * NROW, NROW)]],
                              sl['erows'], sl['semE']).wait()
        pltpu.make_async_copy(msum.at[pl.ds((base + b) * NROW, NROW)],
                              sl['msb'], sl['semS']).wait()

    def compute(b, sl):
        erows_v, msb_v = sl['erows'], sl['msb']

        @plsc.parallel_loop(0, NROW, unroll=8)
        def _(i):
            for k in range(KCH):
                sl2 = pl.ds(k * NLANE, NLANE)
                erows_v[i, sl2] = ((erows_v[i, sl2] + msb_v[i, sl2])
                                   * jnp.float32(1.0 / (NT + 1)))

        u = tuple(u_v[b, pl.ds(k * NLANE, NLANE)] for k in range(KCH))

        scores_v[pl.ds(48, NLANE)] = jnp.full((NLANE,), -1e30, jnp.float32)

        @plsc.parallel_loop(0, W, unroll=5)
        def _(w):
            t = u[0] * erows_v[w, pl.ds(0, NLANE)]
            for k in range(1, KCH):
                t = t + u[k] * erows_v[w, pl.ds(k * NLANE, NLANE)]
            s = jnp.sum(t) * jnp.float32(0.125)
            plsc.store_scatter(scores_v, [_bc_i(w)], _bc_f(s), mask=lane0)

        sv = [scores_v[pl.ds(k * NLANE, NLANE)] for k in range(KCH)]
        m = jnp.max(jnp.maximum(jnp.maximum(sv[0], sv[1]),
                                jnp.maximum(sv[2], sv[3])))
        ev = [jnp.exp(v - m) for v in sv]
        z = jnp.sum(ev[0] + ev[1] + ev[2] + ev[3])
        invv = _bc_f(jnp.float32(1.0)) / _bc_f(z)
        for k in range(KCH):
            scores_v[pl.ds(k * NLANE, NLANE)] = ev[k] * invv

        zero4 = tuple(jnp.zeros((NLANE,), jnp.float32) for _ in range(KCH))

        @plsc.parallel_loop(0, W, unroll=5, carry=zero4)
        def p(w, acc):
            a = plsc.load_gather(scores_v, [_bc_i(w)])
            return tuple(acc[k] + a * erows_v[w, pl.ds(k * NLANE, NLANE)]
                         for k in range(KCH))

        vout = jnp.zeros((NLANE,), jnp.float32)
        for r in range(1 + NEG):
            t = p[0] * erows_v[W + r, pl.ds(0, NLANE)]
            for k in range(1, KCH):
                t = t + p[k] * erows_v[W + r, pl.ds(k * NLANE, NLANE)]
            vout = jnp.where(iota == r, jnp.sum(t), vout)
        out_v[b, :] = vout

    fire_eb(0, slots[0])
    fire_eb(1, slots[1])

    def gbody(g, carry):
        for j in range(2):
            b = 2 * g + j
            s = slots[j]
            wait_eb(b, s)
            compute(b, s)

            @pl.when(b + 2 < NB)
            def _():
                fire_eb(b + 2, s)
        return carry
    lax.fori_loop(0, NB // 2, gbody, 0)

    pltpu.sync_copy(out_v, out_hbm.at[pl.ds(base, NB)])


def _sc_meta(idx_all, seq_index, seq_elem, seq_meta_emb, item_meta_emb,
             seq_mi, seq_mw, item_mi, item_mw):
    mesh = plsc.VectorSubcoreMesh(core_axis_name="c", subcore_axis_name="s")
    f32, i32 = jnp.float32, jnp.int32
    return pl.kernel(
        _sc_meta_body,
        out_type=(jax.ShapeDtypeStruct((B, DP), f32),
                  jax.ShapeDtypeStruct((B * NROW, D), f32)),
        mesh=mesh,
        scratch_types=[
            pltpu.VMEM((NB,), i32),
            pltpu.VMEM((NB * NT,), i32),
            pltpu.VMEM((NB, DP), f32),
            pltpu.VMEM((NB * NT,), i32),
            pltpu.VMEM((NB * NT,), f32),
            pltpu.VMEM((NB * NT, D), f32),
            pltpu.VMEM((NB * NROW,), i32),
            pltpu.VMEM((NMET,), i32),
            pltpu.VMEM((NMET,), i32),
            pltpu.VMEM((NMET,), f32),
            pltpu.VMEM((NMET, D), f32),
            pltpu.VMEM((NROW, D), f32),
            pltpu.VMEM((NMET,), i32),
            pltpu.VMEM((NMET,), i32),
            pltpu.VMEM((NMET,), f32),
            pltpu.VMEM((NMET, D), f32),
            pltpu.VMEM((NROW, D), f32),
        ] + [pltpu.SemaphoreType.DMA] * 11,
        compiler_params=pltpu.CompilerParams(
            needs_layout_passes=False, use_tc_tiling_on_sc=False),
    )(idx_all, seq_index, seq_elem, seq_meta_emb, item_meta_emb,
      seq_mi, seq_mw, item_mi, item_mw)


def _sc_attn(idx_all, u_tab, item_elem, msum):
    mesh = plsc.VectorSubcoreMesh(core_axis_name="c", subcore_axis_name="s")
    f32, i32 = jnp.float32, jnp.int32
    return pl.kernel(
        _sc_attn_body,
        out_type=jax.ShapeDtypeStruct((B, NLANE), f32),
        mesh=mesh,
        scratch_types=[
            pltpu.VMEM((NB * NROW,), i32),
            pltpu.VMEM((NB, DP), f32),
            pltpu.VMEM((D,), f32),
            pltpu.VMEM((NB, NLANE), f32),
            pltpu.VMEM((NROW, DP), f32),
            pltpu.VMEM((NROW, D), f32),
            pltpu.VMEM((NROW, DP), f32),
            pltpu.VMEM((NROW, D), f32),
        ] + [pltpu.SemaphoreType.DMA] * 4,
        compiler_params=pltpu.CompilerParams(
            needs_layout_passes=False, use_tc_tiling_on_sc=False),
    )(idx_all, u_tab, item_elem, msum)


def _row_major_pad(table_t):
    eye_p = jnp.concatenate(
        [jnp.eye(D, dtype=jnp.float32),
         jnp.zeros((D, DP - D), jnp.float32)], axis=1)
    return jax.lax.dot_general(
        table_t, eye_p, (((0,), (0,)), ((), ())),
        precision=jax.lax.Precision.DEFAULT)


def _loss_tc(dots_ref, o_ref):
    x = dots_ref[:]
    col = lax.broadcasted_iota(jnp.int32, x.shape, 1) % NLANE
    sig = 1.0 / (1.0 + jnp.exp(-x))
    eps = jnp.float32(1e-7)
    pos_terms = jnp.log(jnp.clip(sig, eps, 1.0 - eps))
    neg_terms = jnp.log(jnp.clip(1.0 - sig, eps, 1.0 - eps))
    pos_sum = jnp.sum(jnp.where(col == 0, pos_terms, 0.0))
    neg_sum = jnp.sum(jnp.where((col >= 1) & (col <= NEG), neg_terms, 0.0))
    loss_pos = -pos_sum / B
    loss_neg = -neg_sum / (B * NEG)
    loss = (loss_pos + loss_neg / NEG) * jnp.float32(0.5)
    o_ref[:, :] = jnp.reshape(loss, (1, 1))


def kernel(seq_index, item_indices, target_index, seq_element_emb,
           seq_meta_emb, item_element_emb, item_meta_emb, seq_meta_indices,
           seq_meta_weights, item_meta_indices, item_meta_weights,
           neg_indices):
    i32 = jnp.int32
    idx_all = jnp.concatenate(
        [item_indices.astype(i32), target_index[:, None].astype(i32),
         neg_indices.astype(i32)], axis=1).reshape(-1)
    seq_elem_p = _row_major_pad(seq_element_emb.T)
    item_elem_p = _row_major_pad(item_element_emb.T)
    u_tab, msum = _sc_meta(
        idx_all, seq_index.astype(i32), seq_elem_p, seq_meta_emb,
        item_meta_emb,
        seq_meta_indices.astype(i32).T.reshape(-1),
        seq_meta_weights.T.reshape(-1),
        item_meta_indices.astype(i32).T.reshape(-1),
        item_meta_weights.T.reshape(-1))
    dots = _sc_attn(idx_all, u_tab, item_elem_p, msum)
    loss = pl.pallas_call(
        _loss_tc,
        out_shape=jax.ShapeDtypeStruct((1, 1), jnp.float32),
    )(dots.reshape(B * NLANE // 128, 128))
    return loss.reshape(())


# speedup vs baseline: 1.1331x; 1.1331x over previous
import jax
import jax.numpy as jnp
from jax import lax
from jax.experimental import pallas as pl
from jax.experimental.pallas import tpu as pltpu
from jax.experimental.pallas import tpu_sc as plsc

B = 4096
W = 50
NEG = 5
D = 64
DP = 128
NUM_SEQ = 100000
NUM_ITEM = 1000000
NT = 4
NROW = W + 1 + NEG
NMET = NROW * NT
NLANE = 16
NC, NS = 2, 16
NWORK = NC * NS
NB = B // NWORK
KCH = D // NLANE


def _bc_i(s):
    return jnp.zeros((NLANE,), jnp.int32) + s


def _bc_f(s):
    return jnp.zeros((NLANE,), jnp.float32) + s


def _build_flat_meta_idx(idx_ref, base, fidx_ref, n, stride):
    iota = lax.iota(jnp.int32, NLANE)
    for j in range(n * NT // NLANE):
        f = iota + j * NLANE
        it = plsc.load_gather(idx_ref, [base + lax.shift_right_logical(f, 2)])
        fidx_ref[pl.ds(j * NLANE, NLANE)] = it + lax.bitwise_and(f, 3) * stride


def _chunked_igather(tab_hbm, idx_ref, dst_ref, sem, n, chunk=112):
    descs = []
    for c in range(0, n, chunk):
        m = min(chunk, n - c)
        descs.append(pltpu.async_copy(
            tab_hbm.at[idx_ref.at[pl.ds(c, m)]], dst_ref.at[pl.ds(c, m)], sem))
    return descs


def _combine_rows(rows_ref, mrows_ref, mw_ref, n, unroll):
    @plsc.parallel_loop(0, n, unroll=unroll)
    def _(i):
        wts = [plsc.load_gather(mw_ref, [_bc_i(i * NT + t)]) for t in range(NT)]
        for k in range(KCH):
            sl = pl.ds(k * NLANE, NLANE)
            acc = rows_ref[i, sl]
            for t in range(NT):
                acc = acc + wts[t] * mrows_ref[i * NT + t, sl]
            rows_ref[i, sl] = acc * jnp.float32(1.0 / (NT + 1))


def _sc_body(idx_all, seq_index, seq_elem, seq_meta_emb, item_elem,
             item_meta_emb, seq_mi, seq_mw, item_mi, item_mw,
             out_hbm,
             sidx_v, sfidx_v, srows_v, smi_v, smw_v, smrows_v,
             idxa_v, scores_v, out_v,
             ifidx0, erows0, mi0, mw0, mrows0, eidx0,
             ifidx1, erows1, mi1, mw1, mrows1, eidx1,
             sem1, sem2, sem3,
             semE0, semI0, semW0, semM0, semE1, semI1, semW1, semM1):
    wid = lax.axis_index("s") * NC + lax.axis_index("c")
    base = wid * NB
    iota = lax.iota(jnp.int32, NLANE)
    lane0 = iota == 0

    pltpu.sync_copy(seq_index.at[pl.ds(base, NB)], sidx_v)
    c_e = pltpu.async_copy(seq_elem.at[sidx_v], srows_v, sem1)
    _build_flat_meta_idx(sidx_v, 0, sfidx_v, NB, NUM_SEQ)
    dmi = _chunked_igather(seq_mi, sfidx_v, smi_v, sem2, NB * NT, 128)
    dmw = _chunked_igather(seq_mw, sfidx_v, smw_v, sem2, NB * NT, 128)
    for dcp in dmi:
        dcp.wait()
    dmr = _chunked_igather(seq_meta_emb, smi_v, smrows_v, sem3, NB * NT, 128)
    for dcp in dmw + dmr:
        dcp.wait()
    c_e.wait()
    _combine_rows(srows_v, smrows_v, smw_v, NB, 8)

    pltpu.sync_copy(idx_all.at[pl.ds(base * NROW, NB * NROW)],
                    idxa_v.at[pl.ds(0, NB * NROW)])

    slots = (
        dict(ifidx=ifidx0, erows=erows0, mi=mi0, mw=mw0, mrows=mrows0,
             eidx=eidx0, semE=semE0, semI=semI0, semW=semW0, semM=semM0),
        dict(ifidx=ifidx1, erows=erows1, mi=mi1, mw=mw1, mrows=mrows1,
             eidx=eidx1, semE=semE1, semI=semI1, semW=semW1, semM=semM1),
    )

    def fire_mi(b, sl):
        _build_flat_meta_idx(idxa_v, b * NROW, sl['ifidx'], NROW, NUM_ITEM)
        _chunked_igather(item_mi, sl['ifidx'], sl['mi'], sl['semI'], NMET)

    def wait_mi(sl):
        for c in range(0, NMET, 112):
            pltpu.make_async_copy(
                item_mi.at[sl['ifidx'].at[pl.ds(c, 112)]],
                sl['mi'].at[pl.ds(c, 112)], sl['semI']).wait()

    def fire_mw(sl):
        _chunked_igather(item_mw, sl['ifidx'], sl['mw'], sl['semW'], NMET)

    def wait_mw(sl):
        for c in range(0, NMET, 112):
            pltpu.make_async_copy(
                item_mw.at[sl['ifidx'].at[pl.ds(c, 112)]],
                sl['mw'].at[pl.ds(c, 112)], sl['semW']).wait()

    def fire_mrows(sl):
        _chunked_igather(item_meta_emb, sl['mi'], sl['mrows'], sl['semM'], NMET)

    def wait_mrows(sl):
        for c in range(0, NMET, 112):
            pltpu.make_async_copy(
                item_meta_emb.at[sl['mi'].at[pl.ds(c, 112)]],
                sl['mrows'].at[pl.ds(c, 112)], sl['semM']).wait()

    def fire_erows(b, sl):
        for c in range(NROW // NLANE + 1):
            e2 = plsc.load_gather(idxa_v, [b * NROW + iota + c * NLANE]) * 2
            sl['eidx'][pl.ds(c * NLANE, NLANE)] = e2
        pltpu.async_copy(item_elem.at[sl['eidx'].at[pl.ds(0, NROW)]],
                         sl['erows'], sl['semE'])

    def wait_erows(b, sl):
        pltpu.make_async_copy(item_elem.at[sl['eidx'].at[pl.ds(0, NROW)]],
                              sl['erows'], sl['semE']).wait()

    def compute(b, sl):
        erows_v, mrows_v, mw_v = sl['erows'], sl['mrows'], sl['mw']
        _combine_rows(erows_v, mrows_v, mw_v, NROW, 8)

        u = tuple(srows_v[b, pl.ds(k * NLANE, NLANE)] for k in range(KCH))

        scores_v[pl.ds(48, NLANE)] = jnp.full((NLANE,), -1e30, jnp.float32)

        @plsc.parallel_loop(0, W, unroll=5)
        def _(w):
            t = u[0] * erows_v[w, pl.ds(0, NLANE)]
            for k in range(1, KCH):
                t = t + u[k] * erows_v[w, pl.ds(k * NLANE, NLANE)]
            s = jnp.sum(t) * jnp.float32(0.125)
            plsc.store_scatter(scores_v, [_bc_i(w)], _bc_f(s), mask=lane0)

        sv = [scores_v[pl.ds(k * NLANE, NLANE)] for k in range(KCH)]
        m = jnp.max(jnp.maximum(jnp.maximum(sv[0], sv[1]),
                                jnp.maximum(sv[2], sv[3])))
        ev = [jnp.exp(v - m) for v in sv]
        z = jnp.sum(ev[0] + ev[1] + ev[2] + ev[3])
        invv = _bc_f(jnp.float32(1.0)) / _bc_f(z)
        for k in range(KCH):
            scores_v[pl.ds(k * NLANE, NLANE)] = ev[k] * invv

        zero4 = tuple(jnp.zeros((NLANE,), jnp.float32) for _ in range(KCH))

        @plsc.parallel_loop(0, W, unroll=5, carry=zero4)
        def p(w, acc):
            a = plsc.load_gather(scores_v, [_bc_i(w)])
            return tuple(acc[k] + a * erows_v[w, pl.ds(k * NLANE, NLANE)]
                         for k in range(KCH))

        vout = jnp.zeros((NLANE,), jnp.float32)
        for r in range(1 + NEG):
            t = p[0] * erows_v[W + r, pl.ds(0, NLANE)]
            for k in range(1, KCH):
                t = t + p[k] * erows_v[W + r, pl.ds(k * NLANE, NLANE)]
            vout = jnp.where(iota == r, jnp.sum(t), vout)
        out_v[b, :] = vout

    fire_mi(0, slots[0])
    fire_erows(0, slots[0])
    fire_mw(slots[0])
    fire_mi(1, slots[1])
    fire_erows(1, slots[1])
    fire_mw(slots[1])
    wait_mi(slots[0])
    fire_mrows(slots[0])

    def gbody(g, carry):
        for j in range(2):
            b = 2 * g + j
            s, o = slots[j], slots[1 - j]

            @pl.when(b + 1 < NB)
            def _():
                wait_mi(o)
                fire_mrows(o)

            wait_mw(s)
            wait_erows(b, s)

            @pl.when(b + 2 < NB)
            def _():
                fire_mi(b + 2, s)

            wait_mrows(s)
            compute(b, s)

            @pl.when(b + 2 < NB)
            def _():
                fire_erows(b + 2, s)
                fire_mw(s)
        return carry
    lax.fori_loop(0, NB // 2, gbody, 0)

    pltpu.sync_copy(out_v, out_hbm.at[pl.ds(base, NB)])


def _sc_dots(idx_all, seq_index, seq_elem, seq_meta_emb, item_elem,
             item_meta_emb, seq_mi, seq_mw, item_mi, item_mw):
    mesh = plsc.VectorSubcoreMesh(core_axis_name="c", subcore_axis_name="s")
    f32, i32 = jnp.float32, jnp.int32
    return pl.kernel(
        _sc_body,
        out_type=jax.ShapeDtypeStruct((B, NLANE), f32),
        mesh=mesh,
        scratch_types=[
            pltpu.VMEM((NB,), i32),
            pltpu.VMEM((NB * NT,), i32),
            pltpu.VMEM((NB, DP), f32),
            pltpu.VMEM((NB * NT,), i32),
            pltpu.VMEM((NB * NT,), f32),
            pltpu.VMEM((NB * NT, D), f32),
            pltpu.VMEM((NB * NROW + NLANE,), i32),
            pltpu.VMEM((D,), f32),
            pltpu.VMEM((NB, NLANE), f32),
            pltpu.VMEM((NMET,), i32),
            pltpu.VMEM((NROW, D), f32),
            pltpu.VMEM((NMET,), i32),
            pltpu.VMEM((NMET,), f32),
            pltpu.VMEM((NMET, D), f32),
            pltpu.VMEM((NROW + NLANE,), i32),
            pltpu.VMEM((NMET,), i32),
            pltpu.VMEM((NROW, D), f32),
            pltpu.VMEM((NMET,), i32),
            pltpu.VMEM((NMET,), f32),
            pltpu.VMEM((NMET, D), f32),
            pltpu.VMEM((NROW + NLANE,), i32),
            pltpu.SemaphoreType.DMA,
            pltpu.SemaphoreType.DMA,
            pltpu.SemaphoreType.DMA,
            pltpu.SemaphoreType.DMA,
            pltpu.SemaphoreType.DMA,
            pltpu.SemaphoreType.DMA,
            pltpu.SemaphoreType.DMA,
            pltpu.SemaphoreType.DMA,
            pltpu.SemaphoreType.DMA,
            pltpu.SemaphoreType.DMA,
            pltpu.SemaphoreType.DMA,
        ],
        compiler_params=pltpu.CompilerParams(
            needs_layout_passes=False, use_tc_tiling_on_sc=False),
    )(idx_all, seq_index, seq_elem, seq_meta_emb, item_elem,
      item_meta_emb, seq_mi, seq_mw, item_mi, item_mw)


def _row_major_pad(table_t):
    eye_p = jnp.concatenate(
        [jnp.eye(D, dtype=jnp.float32),
         jnp.zeros((D, DP - D), jnp.float32)], axis=1)
    return jax.lax.dot_general(
        table_t, eye_p, (((0,), (0,)), ((), ())),
        precision=jax.lax.Precision.DEFAULT)


def _loss_tc(dots_ref, o_ref):
    x = dots_ref[:]
    col = lax.broadcasted_iota(jnp.int32, x.shape, 1) % NLANE
    sig = 1.0 / (1.0 + jnp.exp(-x))
    eps = jnp.float32(1e-7)
    pos_terms = jnp.log(jnp.clip(sig, eps, 1.0 - eps))
    neg_terms = jnp.log(jnp.clip(1.0 - sig, eps, 1.0 - eps))
    pos_sum = jnp.sum(jnp.where(col == 0, pos_terms, 0.0))
    neg_sum = jnp.sum(jnp.where((col >= 1) & (col <= NEG), neg_terms, 0.0))
    loss_pos = -pos_sum / B
    loss_neg = -neg_sum / (B * NEG)
    loss = (loss_pos + loss_neg / NEG) * jnp.float32(0.5)
    o_ref[:, :] = jnp.reshape(loss, (1, 1))


def kernel(seq_index, item_indices, target_index, seq_element_emb,
           seq_meta_emb, item_element_emb, item_meta_emb, seq_meta_indices,
           seq_meta_weights, item_meta_indices, item_meta_weights,
           neg_indices):
    i32 = jnp.int32
    idx_all = jnp.concatenate(
        [item_indices.astype(i32), target_index[:, None].astype(i32),
         neg_indices.astype(i32)], axis=1).reshape(-1)
    seq_elem_p = _row_major_pad(seq_element_emb.T)
    item_elem_p = _row_major_pad(item_element_emb.T).reshape(2 * NUM_ITEM, D)
    dots = _sc_dots(idx_all, seq_index.astype(i32), seq_elem_p,
                    seq_meta_emb, item_elem_p, item_meta_emb,
                    seq_meta_indices.astype(i32).T.reshape(-1),
                    seq_meta_weights.T.reshape(-1),
                    item_meta_indices.astype(i32).T.reshape(-1),
                    item_meta_weights.T.reshape(-1))
    loss = pl.pallas_call(
        _loss_tc,
        out_shape=jax.ShapeDtypeStruct((1, 1), jnp.float32),
    )(dots.reshape(B * NLANE // 128, 128))
    return loss.reshape(())
